# baseline (speedup 1.0000x reference)
import functools

import jax
import jax.numpy as jnp
import numpy as np
from jax import lax
from jax.experimental import pallas as pl
from jax.experimental.pallas import tpu as pltpu
from jax.experimental.pallas import tpu_sc as plsc

NUM_NODE = 100000
DIM = 64
B = 256
L = 50
N_SAMPLE = 12
ALPHA = 0.2
BETA = 0.005
NEG = -9e15

NC, NS = 2, 16
NW = NC * NS
BL = B * L
NBR = BL * N_SAMPLE
SPAD = 16

_k1, _k2 = jax.random.split(jax.random.key(1234))
_PB = np.asarray(jax.random.permutation(_k1, B))
_PL = np.asarray(jax.random.permutation(_k2, L))
_PB_MAT = np.zeros((B, B), np.float32)
_PB_MAT[np.arange(B), _PB] = 1.0
_M_SUM = np.zeros((L * DIM, DIM), np.float32)
for _l in range(L):
    _M_SUM[_l * DIM + np.arange(DIM), np.arange(DIM)] = 1.0


def _leaky(x, slope):
    return jnp.where(x >= 0, x, slope * x)


def _sc_gather_level1(embedding, adjp, numwp, idx_in, idx_item, idx_tgt):
    n_per_w = BL // NW
    ch = 80
    n_ch = n_per_w // ch
    t_per_w = B // NW
    mesh = plsc.VectorSubcoreMesh(core_axis_name="c", subcore_axis_name="s")

    @functools.partial(
        pl.kernel,
        mesh=mesh,
        compiler_params=pltpu.CompilerParams(use_tc_tiling_on_sc=False),
        out_type=(
            jax.ShapeDtypeStruct((BL, DIM), jnp.float32),
            jax.ShapeDtypeStruct((BL, SPAD), jnp.int32),
            jax.ShapeDtypeStruct((BL, SPAD), jnp.float32),
            jax.ShapeDtypeStruct((BL, DIM), jnp.float32),
            jax.ShapeDtypeStruct((B, DIM), jnp.float32),
        ),
        scratch_types=[
            pltpu.VMEM((ch,), jnp.int32),
            pltpu.VMEM((ch, DIM), jnp.float32),
            pltpu.VMEM((ch, SPAD), jnp.int32),
            pltpu.VMEM((ch, SPAD), jnp.float32),
            pltpu.VMEM((t_per_w,), jnp.int32),
            pltpu.VMEM((t_per_w, DIM), jnp.float32),
            pltpu.SemaphoreType.DMA,
        ],
    )
    def k(emb_hbm, adj_hbm, nw_hbm, iin_hbm, iit_hbm, itg_hbm,
          h_hbm, samp_hbm, wn_hbm, item_hbm, t1_hbm,
          idx_v, rows_v, samp_v, wn_v, tidx_v, trows_v, sem):
        wid = lax.axis_index("s") * NC + lax.axis_index("c")
        base0 = wid * n_per_w

        @pl.loop(0, n_ch)
        def _(c):
            base = base0 + c * ch
            sl = pl.ds(base, ch)
            pltpu.sync_copy(iin_hbm.at[sl], idx_v)
            pltpu.async_copy(emb_hbm.at[idx_v], rows_v, sem).wait()
            pltpu.sync_copy(rows_v, h_hbm.at[sl])
            pltpu.async_copy(adj_hbm.at[idx_v], samp_v, sem).wait()
            pltpu.sync_copy(samp_v, samp_hbm.at[sl])
            pltpu.async_copy(nw_hbm.at[idx_v], wn_v, sem).wait()
            pltpu.sync_copy(wn_v, wn_hbm.at[sl])
            pltpu.sync_copy(iit_hbm.at[sl], idx_v)
            pltpu.async_copy(emb_hbm.at[idx_v], rows_v, sem).wait()
            pltpu.sync_copy(rows_v, item_hbm.at[sl])

        tsl = pl.ds(wid * t_per_w, t_per_w)
        pltpu.sync_copy(itg_hbm.at[tsl], tidx_v)
        pltpu.async_copy(emb_hbm.at[tidx_v], trows_v, sem).wait()
        pltpu.sync_copy(trows_v, t1_hbm.at[tsl])

    return k(embedding, adjp, numwp, idx_in, idx_item, idx_tgt)


def _sc_gather_level2(embedding, nbr_idx):
    n_per_w = NBR // NW
    ch = 120
    n_ch = n_per_w // ch
    mesh = plsc.VectorSubcoreMesh(core_axis_name="c", subcore_axis_name="s")

    @functools.partial(
        pl.kernel,
        mesh=mesh,
        compiler_params=pltpu.CompilerParams(use_tc_tiling_on_sc=False),
        out_type=jax.ShapeDtypeStruct((NBR, DIM), jnp.float32),
        scratch_types=[
            pltpu.VMEM((ch,), jnp.int32),
            pltpu.VMEM((ch, DIM), jnp.float32),
            pltpu.SemaphoreType.DMA,
        ],
    )
    def k(emb_hbm, idx_hbm, out_hbm, idx_v, rows_v, sem):
        wid = lax.axis_index("s") * NC + lax.axis_index("c")
        base0 = wid * n_per_w

        @pl.loop(0, n_ch)
        def _(c):
            sl = pl.ds(base0 + c * ch, ch)
            pltpu.sync_copy(idx_hbm.at[sl], idx_v)
            pltpu.async_copy(emb_hbm.at[idx_v], rows_v, sem).wait()
            pltpu.sync_copy(rows_v, out_hbm.at[sl])

    return k(embedding, nbr_idx)


_BB = 8


def _dense_body(h_ref, adj_ref, mask_ref, t1_ref, wn_ref, nbr_ref, item_ref,
                acat_ref, taw_ref, tav_ref, gw1_ref, gw2_ref, gw3_ref,
                out_ref, hl_ref, hg_ref):
    f32 = jnp.float32
    mask3 = mask_ref[...]
    te = jnp.dot(t1_ref[...], taw_ref[DIM:2 * DIM, :],
                 preferred_element_type=f32)
    item = item_ref[...]
    sess = jnp.sum(item * mask3, axis=1) / jnp.sum(mask3, axis=1)

    for b in range(_BB):
        hb = h_ref[b]
        es = []
        for k4 in range(4):
            hs = hb * acat_ref[k4:k4 + 1, :]
            es.append(lax.dot_general(hs, hb, (((1,), (1,)), ((), ())),
                                      preferred_element_type=f32))
        adjb = adj_ref[b]
        att = jnp.full((L, L), NEG, f32)
        att = jnp.where(adjb == 1, _leaky(es[0], ALPHA), att)
        att = jnp.where(adjb == 2, _leaky(es[1], ALPHA), att)
        att = jnp.where(adjb == 3, _leaky(es[2], ALPHA), att)
        att = jnp.where(adjb == 4, _leaky(es[3], ALPHA), att)
        att = jax.nn.softmax(att, axis=-1)
        hl_b = jnp.dot(att, hb, preferred_element_type=f32)

        e = _leaky(jnp.dot(hb, taw_ref[0:DIM, :], preferred_element_type=f32)
                   + te[b][None, :], ALPHA)
        score = jnp.sum(e * tav_ref[...], axis=-1, keepdims=True)
        score = jnp.where(mask3[b] > 0, score, NEG)
        alpha = jax.nn.softmax(score, axis=0)
        ht_b = alpha * hb

        sb = sess[b][None, :]
        nbrb = nbr_ref[b]
        wnb = wn_ref[b]
        scs = []
        for s in range(N_SAMPLE):
            ex = sb * nbrb[s]
            p = jnp.dot(ex, gw1_ref[0:DIM, :], preferred_element_type=f32)
            p = p + wnb[:, s:s + 1] * gw1_ref[DIM:DIM + 1, :]
            p = _leaky(p, 0.2)
            scs.append(jnp.sum(p * gw2_ref[...], axis=-1, keepdims=True))
        att2 = jax.nn.softmax(jnp.concatenate(scs, axis=1), axis=-1)
        neigh = att2[:, 0:1] * nbrb[0]
        for s in range(1, N_SAMPLE):
            neigh = neigh + att2[:, s:s + 1] * nbrb[s]
        hg_b = jnp.maximum(
            jnp.dot(hb, gw3_ref[0:DIM, :], preferred_element_type=f32)
            + jnp.dot(neigh, gw3_ref[DIM:2 * DIM, :], preferred_element_type=f32),
            0.0)

        out_ref[b, 0:L, :] = ht_b
        out_ref[b, L:2 * L, :] = hl_b + hg_b
        hl_ref[b] = hl_b
        hg_ref[b] = hg_b


def _loss_body(hl_ref, hg_ref, pb_ref, msum_ref, out_ref):
    f32 = jnp.float32
    hl2 = hl_ref[...]
    hg2 = hg_ref[...]
    c1 = jnp.dot(pb_ref[...], hl2, preferred_element_type=f32)
    c2 = jnp.concatenate(
        [c1[:, int(_PL[l]) * DIM:(int(_PL[l]) + 1) * DIM] for l in range(L)],
        axis=1)
    pos = jnp.dot(hl2 * hg2, msum_ref[...], preferred_element_type=f32)
    neg = jnp.dot(hg2 * c2, msum_ref[...], preferred_element_type=f32)
    spos = jax.nn.sigmoid(pos)
    sneg = jax.nn.sigmoid(neg)
    total = jnp.sum(-jnp.log(1e-8 + spos) - jnp.log(1e-8 + (1.0 - sneg)))
    out_ref[...] = total[None, None]


def kernel(inputs, adj, mask_item, item, targets, adj_all, num_w, embedding,
           la_a0, la_a1, la_a2, la_a3, ta_w, ta_v, ga_w1, ga_w2, ga_w3):
    idx_in = inputs.reshape(-1).astype(jnp.int32)
    idx_item = item.reshape(-1).astype(jnp.int32)
    idx_tgt = targets.astype(jnp.int32)
    adjp = jnp.pad(adj_all.astype(jnp.int32), ((0, 0), (0, SPAD - N_SAMPLE)))
    numwp = jnp.pad(num_w, ((0, 0), (0, SPAD - N_SAMPLE)))

    h_flat, samp, wn, item_flat, t1 = _sc_gather_level1(
        embedding, adjp, numwp, idx_in, idx_item, idx_tgt)

    nbr_idx = samp[:, :N_SAMPLE].reshape(B, L, N_SAMPLE)
    nbr_idx = nbr_idx.transpose(0, 2, 1).reshape(-1)
    nbr_flat = _sc_gather_level2(embedding, nbr_idx)

    h = h_flat.reshape(B, L, DIM)
    item_emb = item_flat.reshape(B, L, DIM)
    nbr = nbr_flat.reshape(B, N_SAMPLE, L, DIM)
    wn3 = wn[:, :N_SAMPLE].reshape(B, L, N_SAMPLE)
    mask3 = mask_item.astype(jnp.float32).reshape(B, L, 1)
    acat = jnp.concatenate([la_a0, la_a1, la_a2, la_a3], axis=1).T
    tav = ta_v.T
    gw2 = ga_w2.T

    grid = B // _BB
    out, hl, hg = pl.pallas_call(
        _dense_body,
        grid=(grid,),
        in_specs=[
            pl.BlockSpec((_BB, L, DIM), lambda i: (i, 0, 0)),
            pl.BlockSpec((_BB, L, L), lambda i: (i, 0, 0)),
            pl.BlockSpec((_BB, L, 1), lambda i: (i, 0, 0)),
            pl.BlockSpec((_BB, DIM), lambda i: (i, 0)),
            pl.BlockSpec((_BB, L, N_SAMPLE), lambda i: (i, 0, 0)),
            pl.BlockSpec((_BB, N_SAMPLE, L, DIM), lambda i: (i, 0, 0, 0)),
            pl.BlockSpec((_BB, L, DIM), lambda i: (i, 0, 0)),
            pl.BlockSpec((4, DIM), lambda i: (0, 0)),
            pl.BlockSpec((2 * DIM, DIM), lambda i: (0, 0)),
            pl.BlockSpec((1, DIM), lambda i: (0, 0)),
            pl.BlockSpec((DIM + 1, DIM), lambda i: (0, 0)),
            pl.BlockSpec((1, DIM), lambda i: (0, 0)),
            pl.BlockSpec((2 * DIM, DIM), lambda i: (0, 0)),
        ],
        out_specs=[
            pl.BlockSpec((_BB, 2 * L, DIM), lambda i: (i, 0, 0)),
            pl.BlockSpec((_BB, L, DIM), lambda i: (i, 0, 0)),
            pl.BlockSpec((_BB, L, DIM), lambda i: (i, 0, 0)),
        ],
        out_shape=[
            jax.ShapeDtypeStruct((B, 2 * L, DIM), jnp.float32),
            jax.ShapeDtypeStruct((B, L, DIM), jnp.float32),
            jax.ShapeDtypeStruct((B, L, DIM), jnp.float32),
        ],
    )(h, adj, mask3, t1, wn3, nbr, item_emb, acat, ta_w, tav, ga_w1, gw2, ga_w3)

    loss = pl.pallas_call(
        _loss_body,
        out_shape=jax.ShapeDtypeStruct((1, 1), jnp.float32),
    )(hl.reshape(B, L * DIM), hg.reshape(B, L * DIM),
      jnp.asarray(_PB_MAT), jnp.asarray(_M_SUM))

    return out, BETA * loss[0, 0]
